# baseline (speedup 1.0000x reference)
import functools

import jax
import jax.numpy as jnp
from jax import lax
from jax.experimental import pallas as pl
from jax.experimental.pallas import tpu as pltpu
from jax.experimental.pallas import tpu_sc as plsc

NC = 2
NS = 16
NW = NC * NS
LANES = 16

D = 64
DM = D + 1
CHUNK = 128
GRPS = CHUNK // LANES
EPS = 1e-15


def _rsqrt(s):
    i = plsc.bitcast(s, jnp.int32)
    i = jnp.int32(0x5F3759DF) - lax.shift_right_arithmetic(i, 1)
    r = plsc.bitcast(i, jnp.float32)
    for _ in range(3):
        r = r * (1.5 - 0.5 * s * r * r)
    return r


def _softplus(c):
    e = jnp.exp(-jnp.abs(c))
    z = e / (e + 2.0)
    z2 = z * z
    p = jnp.float32(1.0 / 9.0)
    p = 1.0 / 7.0 + z2 * p
    p = 1.0 / 5.0 + z2 * p
    p = 1.0 / 3.0 + z2 * p
    p = 1.0 + z2 * p
    return jnp.maximum(c, 0.0) + 2.0 * z * p


def _sc_body(n_lookups, x_hbm, mean_hbm, covar_hbm, outm_hbm, outc_hbm,
             idx_all, mrows, crows, outm_v, outc_v, scale16, tmp_v,
             sem_gm, sem_gc, sem_om, sem_oc):
    per_w = n_lookups // NW
    n_chunks = per_w // CHUNK
    wid = lax.axis_index("s") * NC + lax.axis_index("c")
    base = wid * per_w
    iota = lax.iota(jnp.int32, LANES)

    pltpu.sync_copy(x_hbm.at[pl.ds(base, per_w)], idx_all)

    def start_gather(c, b):
        idxs = idx_all.at[pl.ds(c * CHUNK, CHUNK)]
        pltpu.async_copy(mean_hbm.at[idxs], mrows[b], sem_gm[b])
        pltpu.async_copy(covar_hbm.at[idxs], crows[b], sem_gc[b])

    def wait_gather(b):
        idxs = idx_all.at[pl.ds(0, CHUNK)]
        pltpu.make_async_copy(mean_hbm.at[idxs], mrows[b], sem_gm[b]).wait()
        pltpu.make_async_copy(covar_hbm.at[idxs], crows[b], sem_gc[b]).wait()

    def start_out(c, b):
        off = base + c * CHUNK
        pltpu.async_copy(outm_v[b], outm_hbm.at[pl.ds(off * DM, CHUNK * DM)],
                         sem_om[b])
        pltpu.async_copy(outc_v[b], outc_hbm.at[pl.ds(off * D, CHUNK * D)],
                         sem_oc[b])

    def wait_out(b):
        pltpu.make_async_copy(
            outm_v[b], outm_hbm.at[pl.ds(base * DM, CHUNK * DM)],
            sem_om[b]).wait()
        pltpu.make_async_copy(
            outc_v[b], outc_hbm.at[pl.ds(base * D, CHUNK * D)],
            sem_oc[b]).wait()

    def compute(mr, cr, om, oc):
        def grp(gi, carry):
            row0 = gi * LANES
            for l in range(LANES):
                acc = jnp.zeros((LANES,), jnp.float32)
                for j in range(4):
                    m = mr[row0 + l, pl.ds(j * LANES, LANES)]
                    acc = acc + m * m
                plsc.store_scatter(tmp_v, [iota * LANES + l], acc)
            ssum = tmp_v[pl.ds(0, LANES)]
            for t in range(1, LANES):
                ssum = ssum + tmp_v[pl.ds(t * LANES, LANES)]
            s = jnp.maximum(ssum, EPS)
            r = _rsqrt(s)
            n = s * r
            e = jnp.exp(n)
            ei = 1.0 / e
            cosh = 0.5 * (e + ei)
            scale = (0.5 * (e - ei)) * r
            plsc.store_scatter(om, [(row0 + iota) * DM], cosh)
            for l in range(LANES):
                splat = jnp.full((LANES,), scale[l])
                lrow = row0 + l
                for j in range(4):
                    m = mr[lrow, pl.ds(j * LANES, LANES)]
                    plsc.store_scatter(
                        om, [lrow * DM + 1 + j * LANES + iota], m * splat)
                    cv = cr[lrow, pl.ds(j * LANES, LANES)]
                    oc[pl.ds(lrow * D + j * LANES, LANES)] = _softplus(cv)
            return carry

        lax.fori_loop(0, GRPS, grp, 0)

    start_gather(0, 0)
    start_gather(1, 1)

    def pair_body(g, carry):
        for b in range(2):
            c = 2 * g + b
            wait_gather(b)

            @pl.when(c >= 2)
            def _():
                wait_out(b)

            compute(mrows[b], crows[b], outm_v[b], outc_v[b])
            start_out(c, b)

            @pl.when(c + 2 < n_chunks)
            def _():
                start_gather(c + 2, b)
        return carry

    lax.fori_loop(0, n_chunks // 2, pair_body, 0)
    wait_out(0)
    wait_out(1)


@functools.partial(jax.jit, static_argnames=("n_lookups",))
def _run(x_flat, mean_table, covar_table, n_lookups):
    per_w = n_lookups // NW
    mesh = plsc.VectorSubcoreMesh(
        core_axis_name="c", subcore_axis_name="s",
        num_cores=NC, num_subcores=NS)
    fn = pl.kernel(
        functools.partial(_sc_body, n_lookups),
        out_type=(
            jax.ShapeDtypeStruct((n_lookups * DM,), jnp.float32),
            jax.ShapeDtypeStruct((n_lookups * D,), jnp.float32),
        ),
        mesh=mesh,
        compiler_params=pltpu.CompilerParams(
            needs_layout_passes=False, use_tc_tiling_on_sc=False),
        scratch_types=[
            pltpu.VMEM((per_w,), jnp.int32),
            [pltpu.VMEM((CHUNK, D), jnp.float32)] * 2,
            [pltpu.VMEM((CHUNK, D), jnp.float32)] * 2,
            [pltpu.VMEM((CHUNK * DM,), jnp.float32)] * 2,
            [pltpu.VMEM((CHUNK * D,), jnp.float32)] * 2,
            pltpu.VMEM((LANES,), jnp.float32),
            pltpu.VMEM((LANES * LANES,), jnp.float32),
            [pltpu.SemaphoreType.DMA] * 2,
            [pltpu.SemaphoreType.DMA] * 2,
            [pltpu.SemaphoreType.DMA] * 2,
            [pltpu.SemaphoreType.DMA] * 2,
        ],
    )
    return fn(x_flat, mean_table, covar_table)


def kernel(x, mean_table, covar_table):
    b, l = x.shape
    n = b * l
    outm, outc = _run(x.reshape(n), mean_table, covar_table, n)
    return outm.reshape(b, l, DM), outc.reshape(b, l, D)
